# Initial kernel scaffold; baseline (speedup 1.0000x reference)
import jax
import jax.numpy as jnp
from jax.experimental import pallas as pl


def kernel(x, W_embed):
    raise NotImplementedError("write your pallas kernel here")


# speedup vs baseline: 1.1124x; 1.1124x over previous
import functools

import jax
import jax.numpy as jnp
from jax import lax
from jax.experimental import pallas as pl
from jax.experimental.pallas import tpu as pltpu
from jax.experimental.pallas import tpu_sc as plsc

VOCAB = 1000000
EMBED = 32
N = 16384
T = 50
B = N * T

_info = plsc.get_sparse_core_info()
NC, NS = _info.num_cores, _info.num_subcores
NW = NC * NS

CH = 128
B_PER_W = B // NW
NCH_W = B_PER_W // CH
K = 5
SUP = K * CH
NSUP = NCH_W // K


def _embed_body(idx_hbm, table_hbm, out_hbm, idx_v, rows_v, sem0, sem1):
    sems = (sem0, sem1)
    wid = lax.axis_index("s") * NC + lax.axis_index("c")
    cbase = wid * NCH_W
    rbase = wid * B_PER_W

    pltpu.sync_copy(idx_hbm.at[pl.ds(cbase, NCH_W)], idx_v)

    def fire(s, h):
        for k in range(K):
            j = s * K + k
            pltpu.async_copy(
                table_hbm.at[idx_v.at[j]],
                rows_v.at[h, pl.ds(k * CH, CH)],
                sems[h],
            )

    def drain(s, h):
        for k in range(K):
            j = s * K + k
            pltpu.make_async_copy(
                table_hbm.at[idx_v.at[j]],
                rows_v.at[h, pl.ds(k * CH, CH)],
                sems[h],
            ).wait()

    def out_copy(s, h):
        pltpu.sync_copy(rows_v.at[h], out_hbm.at[pl.ds(rbase + s * SUP, SUP)])

    fire(0, 0)
    fire(1, 1)

    @pl.loop(0, (NSUP - 2) // 2)
    def _steady(g):
        for h in range(2):
            s = g * 2 + h
            drain(s, h)
            out_copy(s, h)
            fire(s + 2, h)

    for h in range(2):
        s = NSUP - 2 + h
        drain(s, h)
        out_copy(s, h)


@jax.jit
def kernel(x, W_embed):
    idx = x.reshape(NW * NCH_W, CH).astype(jnp.int32)
    mesh = plsc.VectorSubcoreMesh(core_axis_name="c", subcore_axis_name="s")
    fn = pl.kernel(
        _embed_body,
        out_type=jax.ShapeDtypeStruct((B, EMBED), jnp.float32),
        mesh=mesh,
        scratch_types=[
            pltpu.VMEM((NCH_W, CH), jnp.int32),
            pltpu.VMEM((2, SUP, EMBED), jnp.float32),
            pltpu.SemaphoreType.DMA,
            pltpu.SemaphoreType.DMA,
        ],
        compiler_params=pltpu.CompilerParams(use_tc_tiling_on_sc=False),
    )
    out = fn(idx, W_embed)
    return out.reshape(N, T, EMBED)


# speedup vs baseline: 1.7996x; 1.6178x over previous
import jax
import jax.numpy as jnp
from jax import lax
from jax.experimental import pallas as pl
from jax.experimental.pallas import tpu as pltpu
from jax.experimental.pallas import tpu_sc as plsc

VOCAB = 1000000
EMBED = 32
N = 16384
T = 50

_info = plsc.get_sparse_core_info()
NC, NS = _info.num_cores, _info.num_subcores
NW = NC * NS
N_PER_W = N // NW
SUPN = 8
NSUP = N_PER_W // SUPN


def _embed_body(x_hbm, table_hbm, out_hbm, idx_v, rows_v, sem0, sem1):
    sems = (sem0, sem1)
    wid = lax.axis_index("s") * NC + lax.axis_index("c")
    nbase = wid * N_PER_W

    pltpu.sync_copy(x_hbm.at[pl.ds(nbase, N_PER_W)], idx_v)

    def fire(s, h):
        for k in range(SUPN):
            pltpu.async_copy(
                table_hbm.at[idx_v.at[s * SUPN + k]],
                rows_v.at[h, k],
                sems[h],
            )

    def drain(s, h):
        for k in range(SUPN):
            pltpu.make_async_copy(
                table_hbm.at[idx_v.at[s * SUPN + k]],
                rows_v.at[h, k],
                sems[h],
            ).wait()

    def out_copy(s, h):
        pltpu.sync_copy(
            rows_v.at[h], out_hbm.at[pl.ds(nbase + s * SUPN, SUPN)]
        )

    fire(0, 0)
    fire(1, 1)

    @pl.loop(0, (NSUP - 2) // 2)
    def _steady(g):
        for h in range(2):
            s = g * 2 + h
            drain(s, h)
            out_copy(s, h)
            fire(s + 2, h)

    for h in range(2):
        s = NSUP - 2 + h
        drain(s, h)
        out_copy(s, h)


def kernel(x, W_embed):
    mesh = plsc.VectorSubcoreMesh(core_axis_name="c", subcore_axis_name="s")
    fn = pl.kernel(
        _embed_body,
        out_type=jax.ShapeDtypeStruct((N, T, EMBED), jnp.float32),
        mesh=mesh,
        scratch_types=[
            pltpu.VMEM((N_PER_W, T), jnp.int32),
            pltpu.VMEM((2, SUPN, T, EMBED), jnp.float32),
            pltpu.SemaphoreType.DMA,
            pltpu.SemaphoreType.DMA,
        ],
        compiler_params=pltpu.CompilerParams(use_tc_tiling_on_sc=False),
    )
    return fn(x.astype(jnp.int32), W_embed)
